# baseline (speedup 1.0000x reference)
import jax
import jax.numpy as jnp
from jax.experimental import pallas as pl
from jax.experimental.pallas import tpu as pltpu

_BLOCK_BYTES_TARGET = 2 * 1024 * 1024
_VMEM_BYTES = 64 * 1024 * 1024


def _se_block(x_ref, w1s_ref, b1_ref, w2t_ref, b2_ref, o_ref):
    x = x_ref[...]
    pooled = jnp.sum(x.astype(jnp.float32), axis=-1)
    h = jnp.dot(pooled, w1s_ref[...], preferred_element_type=jnp.float32)
    h = jnp.maximum(h + b1_ref[...], 0.0)
    g = jnp.dot(h, w2t_ref[...], preferred_element_type=jnp.float32)
    g = jax.nn.sigmoid(g + b2_ref[...])
    o_ref[...] = x * g[:, :, None].astype(x.dtype)


def _group_size(batch, block_bytes):
    cap = max(1, _BLOCK_BYTES_TARGET // max(block_bytes, 1))
    nb = 1
    for d in range(1, min(batch, cap) + 1):
        if batch % d == 0:
            nb = d
    return nb


def kernel(x_nchw, w1, b1, w2, b2):
    B, C, H, W = x_nchw.shape
    HW = H * W
    Cr = w1.shape[0]
    dtype = x_nchw.dtype
    d_bytes = jnp.dtype(dtype).itemsize

    w1s = (jnp.asarray(w1, jnp.float32) * (1.0 / HW)).T
    w2t = jnp.asarray(w2, jnp.float32).T
    b1r = jnp.asarray(b1, jnp.float32).reshape(1, Cr)
    b2r = jnp.asarray(b2, jnp.float32).reshape(1, C)

    nb = _group_size(B, C * HW * d_bytes)
    grid = B // nb

    x3 = x_nchw.reshape(B, C, HW)
    out3 = pl.pallas_call(
        _se_block,
        out_shape=jax.ShapeDtypeStruct((B, C, HW), dtype),
        grid=(grid,),
        in_specs=[
            pl.BlockSpec((nb, C, HW), lambda i: (i, 0, 0)),
            pl.BlockSpec((C, Cr), lambda i: (0, 0)),
            pl.BlockSpec((1, Cr), lambda i: (0, 0)),
            pl.BlockSpec((Cr, C), lambda i: (0, 0)),
            pl.BlockSpec((1, C), lambda i: (0, 0)),
        ],
        out_specs=pl.BlockSpec((nb, C, HW), lambda i: (i, 0, 0)),
        compiler_params=pltpu.CompilerParams(
            dimension_semantics=("parallel",),
            vmem_limit_bytes=_VMEM_BYTES,
        ),
        cost_estimate=pl.CostEstimate(
            flops=2 * B * C * HW + 4 * B * C * Cr,
            transcendentals=B * C,
            bytes_accessed=2 * B * C * HW * d_bytes,
        ),
    )(x3, w1s, b1r, w2t, b2r)
    return out3.reshape(B, C, H, W)
